# baseline (speedup 1.0000x reference)
import functools

import jax
import jax.numpy as jnp
from jax import lax
from jax.experimental import pallas as pl
from jax.experimental.pallas import tpu as pltpu
from jax.experimental.pallas import tpu_sc as plsc

N = 10000
E = 320000
D_IN, D_HID, D_OUT = 128, 64, 32

NC, NS = 2, 16
NW = NC * NS
CHUNK = 128
CHUNKS = 80
EPT = CHUNKS * CHUNK
E_PAD = NW * EPT
NP = 10240
RPT = NP // NS
HRPT = RPT // 2
DEGC = 16
NBUF = 4
RING_STEPS = CHUNKS // NBUF


def _mesh():
    return plsc.VectorSubcoreMesh(
        core_axis_name="c", subcore_axis_name="s", num_cores=NC, num_subcores=NS
    )


def _make_deg():
    @functools.partial(
        pl.kernel,
        out_type=jax.ShapeDtypeStruct((NC, NP, DEGC), jnp.float32),
        mesh=_mesh(),
        compiler_params=pltpu.CompilerParams(use_tc_tiling_on_sc=False),
        scratch_types=[
            pltpu.VMEM((CHUNKS, CHUNK), jnp.int32),
            pltpu.VMEM((RPT, DEGC), jnp.float32),
            pltpu.VMEM((CHUNK, DEGC), jnp.float32),
            pltpu.VMEM_SHARED((NP, DEGC), jnp.float32),
        ],
    )
    def deg(dst_hbm, out_hbm, dst_v, obuf, ones_v, acc):
        c = lax.axis_index("c")
        s = lax.axis_index("s")
        wid = c * NS + s
        pltpu.sync_copy(dst_hbm.at[wid], dst_v)

        def fill_obuf(i, carry):
            obuf[i] = jnp.ones((DEGC,), jnp.float32)
            return carry

        lax.fori_loop(0, RPT, fill_obuf, 0)

        def fill_ones(i, carry):
            ones_v[i] = jnp.ones((DEGC,), jnp.float32)
            return carry

        lax.fori_loop(0, CHUNK, fill_ones, 0)

        r0 = s * RPT
        pltpu.sync_copy(obuf, acc.at[pl.ds(r0, RPT)])
        plsc.subcore_barrier()

        def body(j, carry):
            pltpu.sync_copy(ones_v, acc.at[dst_v.at[j]], add=True)
            return carry

        lax.fori_loop(0, CHUNKS, body, 0)
        plsc.subcore_barrier()
        pltpu.sync_copy(acc.at[pl.ds(r0, RPT)], obuf)
        pltpu.sync_copy(obuf, out_hbm.at[c, pl.ds(r0, RPT)])

    return deg


def _ring_loop(table, src_v, dst_v, gbuf, acc, gsem, ssem):
    def g_start(j, b):
        pltpu.async_copy(table.at[src_v.at[j]], gbuf.at[b], gsem.at[b])

    def g_wait(j, b):
        pltpu.make_async_copy(table.at[src_v.at[j]], gbuf.at[b],
                              gsem.at[b]).wait()

    def s_start(j, b):
        pltpu.async_copy(gbuf.at[b], acc.at[dst_v.at[j]], ssem.at[b],
                         add=True)

    def s_wait(j, b):
        pltpu.make_async_copy(gbuf.at[b], acc.at[dst_v.at[j]],
                              ssem.at[b]).wait()

    g_start(0, 0)

    def body(g, carry):
        j0 = g * NBUF
        for b in range(NBUF):
            j = j0 + b
            nb = (b + 1) % NBUF
            g_wait(j, b)
            if b == NBUF - 1:
                s_wait(j - (NBUF - 1), nb)
            else:
                @pl.when(g > 0)
                def _():
                    s_wait(j - (NBUF - 1), nb)
            s_start(j, b)
            if b == NBUF - 1:
                @pl.when(g < RING_STEPS - 1)
                def _():
                    g_start(j + 1, nb)
            else:
                g_start(j + 1, nb)
        return carry

    lax.fori_loop(0, RING_STEPS, body, 0)
    for b in range(1, NBUF):
        s_wait(CHUNKS - NBUF + b, b)


def _zero_fill(buf, rows, width):
    def fill(i, carry):
        for k in range(width // 16):
            buf[i, pl.ds(k * 16, 16)] = jnp.zeros((16,), jnp.float32)
        return carry

    lax.fori_loop(0, rows, fill, 0)


def _make_agg(D):
    @functools.partial(
        pl.kernel,
        out_type=jax.ShapeDtypeStruct((NC, NP, D), jnp.float32),
        mesh=_mesh(),
        compiler_params=pltpu.CompilerParams(use_tc_tiling_on_sc=False),
        scratch_types=[
            pltpu.VMEM((CHUNKS, CHUNK), jnp.int32),
            pltpu.VMEM((CHUNKS, CHUNK), jnp.int32),
            pltpu.VMEM((NBUF, CHUNK, D), jnp.float32),
            pltpu.VMEM_SHARED((NP, D), jnp.float32),
            pltpu.SemaphoreType.DMA((NBUF,)),
            pltpu.SemaphoreType.DMA((NBUF,)),
        ],
    )
    def agg(src_hbm, dst_hbm, table_hbm, out_hbm, src_v, dst_v, gbuf,
            acc, gsem, ssem):
        c = lax.axis_index("c")
        s = lax.axis_index("s")
        wid = c * NS + s
        pltpu.sync_copy(src_hbm.at[wid], src_v)
        pltpu.sync_copy(dst_hbm.at[wid], dst_v)
        r0 = s * RPT

        @pl.when(c == 0)
        def _():
            pltpu.sync_copy(table_hbm.at[pl.ds(r0, RPT)],
                            acc.at[pl.ds(r0, RPT)])

        @pl.when(c == 1)
        def _():
            _zero_fill(gbuf.at[0], CHUNK, D)
            for k in range(RPT // CHUNK):
                pltpu.sync_copy(gbuf.at[0],
                                acc.at[pl.ds(r0 + k * CHUNK, CHUNK)])

        plsc.subcore_barrier()
        _ring_loop(table_hbm, src_v, dst_v, gbuf, acc, gsem, ssem)
        plsc.subcore_barrier()
        pltpu.sync_copy(acc.at[pl.ds(r0, RPT)], out_hbm.at[c, pl.ds(r0, RPT)])

    return agg


def _make_agg1():
    D = D_HID

    @functools.partial(
        pl.kernel,
        out_type=[
            jax.ShapeDtypeStruct((NC, NP, D), jnp.float32),
            jax.ShapeDtypeStruct((NC, NP, D), jnp.float32),
            jax.ShapeDtypeStruct((NC, NP, DEGC), jnp.float32),
        ],
        mesh=_mesh(),
        compiler_params=pltpu.CompilerParams(use_tc_tiling_on_sc=False),
        scratch_types=[
            pltpu.VMEM((CHUNKS, CHUNK), jnp.int32),
            pltpu.VMEM((CHUNKS, CHUNK), jnp.int32),
            pltpu.VMEM((NBUF, CHUNK, D), jnp.float32),
            pltpu.VMEM((HRPT, DEGC), jnp.float32),
            pltpu.VMEM((HRPT, DEGC), jnp.float32),
            pltpu.VMEM((HRPT, D), jnp.float32),
            pltpu.VMEM_SHARED((NP, D), jnp.float32),
            pltpu.SemaphoreType.DMA((NBUF,)),
            pltpu.SemaphoreType.DMA((NBUF,)),
        ],
    )
    def agg1(src_hbm, dst_hbm, y_hbm, d_hbm, out_hbm, ht_hbm, dinv_hbm,
             src_v, dst_v, gbuf, dbuf0, dbuf1, ybuf, acc, gsem, ssem):
        c = lax.axis_index("c")
        s = lax.axis_index("s")
        wid = c * NS + s
        pltpu.sync_copy(src_hbm.at[wid], src_v)
        pltpu.sync_copy(dst_hbm.at[wid], dst_v)

        r0 = s * RPT
        for half in range(2):
            h0 = r0 + half * HRPT
            pltpu.sync_copy(d_hbm.at[0, pl.ds(h0, HRPT)], dbuf0)
            pltpu.sync_copy(d_hbm.at[1, pl.ds(h0, HRPT)], dbuf1)
            pltpu.sync_copy(y_hbm.at[pl.ds(h0, HRPT)], ybuf)

            def row(i, carry):
                dsum = dbuf0[i] + dbuf1[i] - 1.0
                xi = 0x5F3759DF - (lax.bitcast_convert_type(dsum, jnp.int32) >> 1)
                xx = lax.bitcast_convert_type(xi, jnp.float32)
                for _ in range(3):
                    xx = xx * (1.5 - 0.5 * dsum * xx * xx)
                dbuf0[i] = xx
                for k in range(D // 16):
                    ybuf[i, pl.ds(k * 16, 16)] = (
                        ybuf[i, pl.ds(k * 16, 16)] * xx)
                return carry

            lax.fori_loop(0, HRPT, row, 0)
            pltpu.sync_copy(dbuf0, dinv_hbm.at[c, pl.ds(h0, HRPT)])
            pltpu.sync_copy(ybuf, ht_hbm.at[c, pl.ds(h0, HRPT)])
            @pl.when(c == 0)
            def _():
                pltpu.sync_copy(ybuf, acc.at[pl.ds(h0, HRPT)])

        @pl.when(c == 1)
        def _():
            _zero_fill(gbuf.at[0], CHUNK, D)
            for k in range(RPT // CHUNK):
                pltpu.sync_copy(gbuf.at[0],
                                acc.at[pl.ds(r0 + k * CHUNK, CHUNK)])

        plsc.subcore_barrier()
        _ring_loop(ht_hbm.at[c], src_v, dst_v, gbuf, acc, gsem, ssem)
        plsc.subcore_barrier()
        pltpu.sync_copy(acc.at[pl.ds(r0, RPT)], out_hbm.at[c, pl.ds(r0, RPT)])

    return agg1


@functools.lru_cache(maxsize=None)
def _get_deg():
    return _make_deg()


@functools.lru_cache(maxsize=None)
def _get_agg(D):
    return _make_agg(D)


@functools.lru_cache(maxsize=None)
def _get_agg1():
    return _make_agg1()


def _tcmm_body(x_ref, w_ref, y_ref):
    y_ref[0:N, :] = jnp.dot(x_ref[...], w_ref[...],
                            preferred_element_type=jnp.float32)
    y_ref[N:NP, :] = jnp.zeros((NP - N, D_HID), jnp.float32)


def _tcmm(x, W1):
    return pl.pallas_call(
        _tcmm_body,
        out_shape=jax.ShapeDtypeStruct((NP, D_HID), jnp.float32),
    )(x, W1)


def _comb_body(p_ref, dinv_ref, b_ref, w_ref, out_ref):
    dinv = dinv_ref[...]
    h = jnp.maximum(dinv * (p_ref[0] + p_ref[1]) + b_ref[...], 0.0)
    out_ref[...] = dinv * jnp.dot(
        h, w_ref[...], preferred_element_type=jnp.float32
    )


def _comb(p, dinv, b, W, D_next):
    return pl.pallas_call(
        _comb_body,
        out_shape=jax.ShapeDtypeStruct((NP, D_next), jnp.float32),
    )(p, dinv, b, W)


def _final_body(p_ref, dinv_ref, b_ref, out_ref):
    dinv = dinv_ref[...]
    h = jnp.maximum(dinv * (p_ref[0] + p_ref[1]) + b_ref[...], 0.0)
    out_ref[...] = jnp.sum(h[0:N], axis=0, keepdims=True) * (1.0 / N)


def _final(p, dinv, b):
    return pl.pallas_call(
        _final_body,
        out_shape=jax.ShapeDtypeStruct((1, D_OUT), jnp.float32),
    )(p, dinv, b)


def kernel(x, edge_index, last_rej_rate, W1, b1, W2, b2, W3, b3):
    pad_dst = N + jnp.arange(E_PAD - E, dtype=jnp.int32) % (NP - N)
    pad_src = N + (jnp.arange(E_PAD - E, dtype=jnp.int32) + 7) % (NP - N)
    srcp = jnp.concatenate([edge_index[0], pad_src]).reshape(
        NW, CHUNKS, CHUNK)
    dstp = jnp.concatenate([edge_index[1], pad_dst]).reshape(
        NW, CHUNKS, CHUNK)
    d = _get_deg()(dstp)
    y1 = _tcmm(x, W1)
    p1, _, dinv_sc = _get_agg1()(srcp, dstp, y1, d)
    dinv = dinv_sc[0, :, 0:1]
    ht2 = _comb(p1, dinv, b1.reshape(1, -1), W2, D_HID)
    p2 = _get_agg(D_HID)(srcp, dstp, ht2)
    ht3 = _comb(p2, dinv, b2.reshape(1, -1), W3, D_OUT)
    p3 = _get_agg(D_OUT)(srcp, dstp, ht3)
    pooled = _final(p3, dinv, b3.reshape(1, -1))
    rej = jnp.reshape(last_rej_rate, (1, 1)).astype(jnp.float32)
    return jnp.concatenate([pooled, rej], axis=-1)


# speedup vs baseline: 1.0037x; 1.0037x over previous
import functools

import jax
import jax.numpy as jnp
from jax import lax
from jax.experimental import pallas as pl
from jax.experimental.pallas import tpu as pltpu
from jax.experimental.pallas import tpu_sc as plsc

N = 10000
E = 320000
D_IN, D_HID, D_OUT = 128, 64, 32

NC, NS = 2, 16
NW = NC * NS
CHUNK = 128
CHUNKS = 80
EPT = CHUNKS * CHUNK
E_PAD = NW * EPT
NP = 10240
RPT = NP // NS
HRPT = RPT // 2
DEGC = 16
NBUF = 4
RING_STEPS = CHUNKS // NBUF


def _mesh():
    return plsc.VectorSubcoreMesh(
        core_axis_name="c", subcore_axis_name="s", num_cores=NC, num_subcores=NS
    )


def _make_deg():
    @functools.partial(
        pl.kernel,
        out_type=jax.ShapeDtypeStruct((NC, NP, DEGC), jnp.float32),
        mesh=_mesh(),
        compiler_params=pltpu.CompilerParams(use_tc_tiling_on_sc=False),
        scratch_types=[
            pltpu.VMEM((CHUNKS, CHUNK), jnp.int32),
            pltpu.VMEM((RPT, DEGC), jnp.float32),
            pltpu.VMEM((CHUNK, DEGC), jnp.float32),
            pltpu.VMEM_SHARED((NP, DEGC), jnp.float32),
        ],
    )
    def deg(dst_hbm, out_hbm, dst_v, obuf, ones_v, acc):
        c = lax.axis_index("c")
        s = lax.axis_index("s")
        wid = c * NS + s
        pltpu.sync_copy(dst_hbm.at[wid], dst_v)

        def fill_obuf(i8, carry):
            for u in range(8):
                obuf[i8 * 8 + u] = jnp.ones((DEGC,), jnp.float32)
            return carry

        lax.fori_loop(0, RPT // 8, fill_obuf, 0)

        def fill_ones(i8, carry):
            for u in range(8):
                ones_v[i8 * 8 + u] = jnp.ones((DEGC,), jnp.float32)
            return carry

        lax.fori_loop(0, CHUNK // 8, fill_ones, 0)

        r0 = s * RPT
        pltpu.sync_copy(obuf, acc.at[pl.ds(r0, RPT)])
        plsc.subcore_barrier()

        def body(j, carry):
            pltpu.sync_copy(ones_v, acc.at[dst_v.at[j]], add=True)
            return carry

        lax.fori_loop(0, CHUNKS, body, 0)
        plsc.subcore_barrier()
        pltpu.sync_copy(acc.at[pl.ds(r0, RPT)], obuf)
        pltpu.sync_copy(obuf, out_hbm.at[c, pl.ds(r0, RPT)])

    return deg


def _ring_loop(table, src_v, dst_v, gbuf, acc, gsem, ssem):
    def g_start(j, b):
        pltpu.async_copy(table.at[src_v.at[j]], gbuf.at[b], gsem.at[b])

    def g_wait(j, b):
        pltpu.make_async_copy(table.at[src_v.at[j]], gbuf.at[b],
                              gsem.at[b]).wait()

    def s_start(j, b):
        pltpu.async_copy(gbuf.at[b], acc.at[dst_v.at[j]], ssem.at[b],
                         add=True)

    def s_wait(j, b):
        pltpu.make_async_copy(gbuf.at[b], acc.at[dst_v.at[j]],
                              ssem.at[b]).wait()

    g_start(0, 0)

    def body(g, carry):
        j0 = g * NBUF
        for b in range(NBUF):
            j = j0 + b
            nb = (b + 1) % NBUF
            g_wait(j, b)
            if b == NBUF - 1:
                s_wait(j - (NBUF - 1), nb)
            else:
                @pl.when(g > 0)
                def _():
                    s_wait(j - (NBUF - 1), nb)
            s_start(j, b)
            if b == NBUF - 1:
                @pl.when(g < RING_STEPS - 1)
                def _():
                    g_start(j + 1, nb)
            else:
                g_start(j + 1, nb)
        return carry

    lax.fori_loop(0, RING_STEPS, body, 0)
    for b in range(1, NBUF):
        s_wait(CHUNKS - NBUF + b, b)


def _zero_fill(buf, rows, width):
    def fill(i4, carry):
        for u in range(4):
            for k in range(width // 16):
                buf[i4 * 4 + u, pl.ds(k * 16, 16)] = (
                    jnp.zeros((16,), jnp.float32))
        return carry

    lax.fori_loop(0, rows // 4, fill, 0)


def _make_agg(D):
    @functools.partial(
        pl.kernel,
        out_type=jax.ShapeDtypeStruct((NC, NP, D), jnp.float32),
        mesh=_mesh(),
        compiler_params=pltpu.CompilerParams(use_tc_tiling_on_sc=False),
        scratch_types=[
            pltpu.VMEM((CHUNKS, CHUNK), jnp.int32),
            pltpu.VMEM((CHUNKS, CHUNK), jnp.int32),
            pltpu.VMEM((NBUF, CHUNK, D), jnp.float32),
            pltpu.VMEM_SHARED((NP, D), jnp.float32),
            pltpu.SemaphoreType.DMA((NBUF,)),
            pltpu.SemaphoreType.DMA((NBUF,)),
        ],
    )
    def agg(src_hbm, dst_hbm, table_hbm, out_hbm, src_v, dst_v, gbuf,
            acc, gsem, ssem):
        c = lax.axis_index("c")
        s = lax.axis_index("s")
        wid = c * NS + s
        pltpu.sync_copy(src_hbm.at[wid], src_v)
        pltpu.sync_copy(dst_hbm.at[wid], dst_v)
        r0 = s * RPT

        @pl.when(c == 0)
        def _():
            pltpu.sync_copy(table_hbm.at[pl.ds(r0, RPT)],
                            acc.at[pl.ds(r0, RPT)])

        @pl.when(c == 1)
        def _():
            _zero_fill(gbuf.at[0], CHUNK, D)
            for k in range(RPT // CHUNK):
                pltpu.sync_copy(gbuf.at[0],
                                acc.at[pl.ds(r0 + k * CHUNK, CHUNK)])

        plsc.subcore_barrier()
        _ring_loop(table_hbm, src_v, dst_v, gbuf, acc, gsem, ssem)
        plsc.subcore_barrier()
        pltpu.sync_copy(acc.at[pl.ds(r0, RPT)], out_hbm.at[c, pl.ds(r0, RPT)])

    return agg


def _make_agg1():
    D = D_HID

    @functools.partial(
        pl.kernel,
        out_type=[
            jax.ShapeDtypeStruct((NC, NP, D), jnp.float32),
            jax.ShapeDtypeStruct((NC, NP, D), jnp.float32),
            jax.ShapeDtypeStruct((NC, NP, DEGC), jnp.float32),
        ],
        mesh=_mesh(),
        compiler_params=pltpu.CompilerParams(use_tc_tiling_on_sc=False),
        scratch_types=[
            pltpu.VMEM((CHUNKS, CHUNK), jnp.int32),
            pltpu.VMEM((CHUNKS, CHUNK), jnp.int32),
            pltpu.VMEM((NBUF, CHUNK, D), jnp.float32),
            pltpu.VMEM((HRPT, DEGC), jnp.float32),
            pltpu.VMEM((HRPT, DEGC), jnp.float32),
            pltpu.VMEM((HRPT, D), jnp.float32),
            pltpu.VMEM_SHARED((NP, D), jnp.float32),
            pltpu.SemaphoreType.DMA((NBUF,)),
            pltpu.SemaphoreType.DMA((NBUF,)),
        ],
    )
    def agg1(src_hbm, dst_hbm, y_hbm, d_hbm, out_hbm, ht_hbm, dinv_hbm,
             src_v, dst_v, gbuf, dbuf0, dbuf1, ybuf, acc, gsem, ssem):
        c = lax.axis_index("c")
        s = lax.axis_index("s")
        wid = c * NS + s
        pltpu.sync_copy(src_hbm.at[wid], src_v)
        pltpu.sync_copy(dst_hbm.at[wid], dst_v)

        r0 = s * RPT
        for half in range(2):
            h0 = r0 + half * HRPT
            pltpu.sync_copy(d_hbm.at[0, pl.ds(h0, HRPT)], dbuf0)
            pltpu.sync_copy(d_hbm.at[1, pl.ds(h0, HRPT)], dbuf1)
            pltpu.sync_copy(y_hbm.at[pl.ds(h0, HRPT)], ybuf)

            def row(i4, carry):
                for u in range(4):
                    i = i4 * 4 + u
                    dsum = dbuf0[i] + dbuf1[i] - 1.0
                    xi = 0x5F3759DF - (
                        lax.bitcast_convert_type(dsum, jnp.int32) >> 1)
                    xx = lax.bitcast_convert_type(xi, jnp.float32)
                    for _ in range(3):
                        xx = xx * (1.5 - 0.5 * dsum * xx * xx)
                    dbuf0[i] = xx
                    for k in range(D // 16):
                        ybuf[i, pl.ds(k * 16, 16)] = (
                            ybuf[i, pl.ds(k * 16, 16)] * xx)
                return carry

            lax.fori_loop(0, HRPT // 4, row, 0)
            pltpu.sync_copy(dbuf0, dinv_hbm.at[c, pl.ds(h0, HRPT)])
            pltpu.sync_copy(ybuf, ht_hbm.at[c, pl.ds(h0, HRPT)])
            @pl.when(c == 0)
            def _():
                pltpu.sync_copy(ybuf, acc.at[pl.ds(h0, HRPT)])

        @pl.when(c == 1)
        def _():
            _zero_fill(gbuf.at[0], CHUNK, D)
            for k in range(RPT // CHUNK):
                pltpu.sync_copy(gbuf.at[0],
                                acc.at[pl.ds(r0 + k * CHUNK, CHUNK)])

        plsc.subcore_barrier()
        _ring_loop(ht_hbm.at[c], src_v, dst_v, gbuf, acc, gsem, ssem)
        plsc.subcore_barrier()
        pltpu.sync_copy(acc.at[pl.ds(r0, RPT)], out_hbm.at[c, pl.ds(r0, RPT)])

    return agg1


@functools.lru_cache(maxsize=None)
def _get_deg():
    return _make_deg()


@functools.lru_cache(maxsize=None)
def _get_agg(D):
    return _make_agg(D)


@functools.lru_cache(maxsize=None)
def _get_agg1():
    return _make_agg1()


def _tcmm_body(x_ref, w_ref, y_ref):
    y_ref[0:N, :] = jnp.dot(x_ref[...], w_ref[...],
                            preferred_element_type=jnp.float32)
    y_ref[N:NP, :] = jnp.zeros((NP - N, D_HID), jnp.float32)


def _tcmm(x, W1):
    return pl.pallas_call(
        _tcmm_body,
        out_shape=jax.ShapeDtypeStruct((NP, D_HID), jnp.float32),
    )(x, W1)


def _comb_body(p_ref, dinv_ref, b_ref, w_ref, out_ref):
    dinv = dinv_ref[...]
    h = jnp.maximum(dinv * (p_ref[0] + p_ref[1]) + b_ref[...], 0.0)
    out_ref[...] = dinv * jnp.dot(
        h, w_ref[...], preferred_element_type=jnp.float32
    )


def _comb(p, dinv, b, W, D_next):
    return pl.pallas_call(
        _comb_body,
        out_shape=jax.ShapeDtypeStruct((NP, D_next), jnp.float32),
    )(p, dinv, b, W)


def _final_body(p_ref, dinv_ref, b_ref, out_ref):
    dinv = dinv_ref[...]
    h = jnp.maximum(dinv * (p_ref[0] + p_ref[1]) + b_ref[...], 0.0)
    out_ref[...] = jnp.sum(h[0:N], axis=0, keepdims=True) * (1.0 / N)


def _final(p, dinv, b):
    return pl.pallas_call(
        _final_body,
        out_shape=jax.ShapeDtypeStruct((1, D_OUT), jnp.float32),
    )(p, dinv, b)


def kernel(x, edge_index, last_rej_rate, W1, b1, W2, b2, W3, b3):
    pad_dst = N + jnp.arange(E_PAD - E, dtype=jnp.int32) % (NP - N)
    pad_src = N + (jnp.arange(E_PAD - E, dtype=jnp.int32) + 7) % (NP - N)
    srcp = jnp.concatenate([edge_index[0], pad_src]).reshape(
        NW, CHUNKS, CHUNK)
    dstp = jnp.concatenate([edge_index[1], pad_dst]).reshape(
        NW, CHUNKS, CHUNK)
    d = _get_deg()(dstp)
    y1 = _tcmm(x, W1)
    p1, _, dinv_sc = _get_agg1()(srcp, dstp, y1, d)
    dinv = dinv_sc[0, :, 0:1]
    ht2 = _comb(p1, dinv, b1.reshape(1, -1), W2, D_HID)
    p2 = _get_agg(D_HID)(srcp, dstp, ht2)
    ht3 = _comb(p2, dinv, b2.reshape(1, -1), W3, D_OUT)
    p3 = _get_agg(D_OUT)(srcp, dstp, ht3)
    pooled = _final(p3, dinv, b3.reshape(1, -1))
    rej = jnp.reshape(last_rej_rate, (1, 1)).astype(jnp.float32)
    return jnp.concatenate([pooled, rej], axis=-1)


# speedup vs baseline: 1.0091x; 1.0053x over previous
import functools

import jax
import jax.numpy as jnp
from jax import lax
from jax.experimental import pallas as pl
from jax.experimental.pallas import tpu as pltpu
from jax.experimental.pallas import tpu_sc as plsc

N = 10000
E = 320000
D_IN, D_HID, D_OUT = 128, 64, 32

NC, NS = 2, 16
NW = NC * NS
CHUNK = 128
CHUNKS = 80
EPT = CHUNKS * CHUNK
E_PAD = NW * EPT
NP = 10240
RPT = NP // NS
DEGC = 16


def _mesh():
    return plsc.VectorSubcoreMesh(
        core_axis_name="c", subcore_axis_name="s", num_cores=NC, num_subcores=NS
    )


def _make_deg():
    @functools.partial(
        pl.kernel,
        out_type=jax.ShapeDtypeStruct((NC, NP, DEGC), jnp.float32),
        mesh=_mesh(),
        compiler_params=pltpu.CompilerParams(use_tc_tiling_on_sc=False),
        scratch_types=[
            pltpu.VMEM((EPT,), jnp.int32),
            pltpu.VMEM((RPT, DEGC), jnp.float32),
            pltpu.VMEM((CHUNK, DEGC), jnp.float32),
            pltpu.VMEM_SHARED((NP, DEGC), jnp.float32),
        ],
    )
    def deg(dst_hbm, out_hbm, dst_v, obuf, ones_v, acc):
        c = lax.axis_index("c")
        s = lax.axis_index("s")
        wid = c * NS + s
        pltpu.sync_copy(dst_hbm.at[pl.ds(wid * EPT, EPT)], dst_v)

        def fill_obuf(i, carry):
            obuf[i] = jnp.ones((DEGC,), jnp.float32)
            return carry

        lax.fori_loop(0, RPT, fill_obuf, 0)

        def fill_ones(i, carry):
            ones_v[i] = jnp.ones((DEGC,), jnp.float32)
            return carry

        lax.fori_loop(0, CHUNK, fill_ones, 0)

        r0 = s * RPT
        pltpu.sync_copy(obuf, acc.at[pl.ds(r0, RPT)])
        plsc.subcore_barrier()

        def body(j, carry):
            pltpu.sync_copy(
                ones_v, acc.at[dst_v.at[pl.ds(j * CHUNK, CHUNK)]], add=True)
            return carry

        lax.fori_loop(0, CHUNKS, body, 0)
        plsc.subcore_barrier()
        pltpu.sync_copy(acc.at[pl.ds(r0, RPT)], obuf)
        pltpu.sync_copy(obuf, out_hbm.at[c, pl.ds(r0, RPT)])

    return deg


NBUF = 4
RING_STEPS = CHUNKS // NBUF


def _make_agg(D):
    @functools.partial(
        pl.kernel,
        out_type=jax.ShapeDtypeStruct((NC, NP, D), jnp.float32),
        mesh=_mesh(),
        compiler_params=pltpu.CompilerParams(use_tc_tiling_on_sc=False),
        scratch_types=[
            pltpu.VMEM((EPT,), jnp.int32),
            pltpu.VMEM((EPT,), jnp.int32),
            pltpu.VMEM((NBUF, CHUNK, D), jnp.float32),
            pltpu.VMEM_SHARED((NP, D), jnp.float32),
            pltpu.SemaphoreType.DMA((NBUF,)),
            pltpu.SemaphoreType.DMA((NBUF,)),
        ],
    )
    def agg(src_hbm, dst_hbm, table_hbm, out_hbm, src_v, dst_v, gbuf,
            acc, gsem, ssem):
        c = lax.axis_index("c")
        s = lax.axis_index("s")
        wid = c * NS + s
        pltpu.sync_copy(src_hbm.at[pl.ds(wid * EPT, EPT)], src_v)
        pltpu.sync_copy(dst_hbm.at[pl.ds(wid * EPT, EPT)], dst_v)

        def g_start(j, b):
            pltpu.async_copy(table_hbm.at[src_v.at[pl.ds(j * CHUNK, CHUNK)]],
                             gbuf.at[b], gsem.at[b])

        def g_wait(j, b):
            pltpu.make_async_copy(
                table_hbm.at[src_v.at[pl.ds(j * CHUNK, CHUNK)]],
                gbuf.at[b], gsem.at[b]).wait()

        def s_start(j, b):
            pltpu.async_copy(gbuf.at[b],
                             acc.at[dst_v.at[pl.ds(j * CHUNK, CHUNK)]],
                             ssem.at[b], add=True)

        def s_wait(j, b):
            pltpu.make_async_copy(
                gbuf.at[b], acc.at[dst_v.at[pl.ds(j * CHUNK, CHUNK)]],
                ssem.at[b]).wait()

        g_start(0, 0)
        r0 = s * RPT
        pltpu.sync_copy(table_hbm.at[pl.ds(r0, RPT)], acc.at[pl.ds(r0, RPT)])
        plsc.subcore_barrier()

        def body(g, carry):
            j0 = g * NBUF
            for b in range(NBUF):
                j = j0 + b
                nb = (b + 1) % NBUF
                g_wait(j, b)
                if b == NBUF - 1:
                    s_wait(j - (NBUF - 1), nb)
                else:
                    @pl.when(g > 0)
                    def _():
                        s_wait(j - (NBUF - 1), nb)
                s_start(j, b)
                if b == NBUF - 1:
                    @pl.when(g < RING_STEPS - 1)
                    def _():
                        g_start(j + 1, nb)
                else:
                    g_start(j + 1, nb)
            return carry

        lax.fori_loop(0, RING_STEPS, body, 0)
        for b in range(1, NBUF):
            s_wait(CHUNKS - NBUF + b, b)
        plsc.subcore_barrier()
        pltpu.sync_copy(acc.at[pl.ds(r0, RPT)], out_hbm.at[c, pl.ds(r0, RPT)])

    return agg


@functools.lru_cache(maxsize=None)
def _get_deg():
    return _make_deg()


@functools.lru_cache(maxsize=None)
def _get_agg(D):
    return _make_agg(D)


def _tcmm_body(x_ref, w_ref, y_ref):
    y_ref[0:N, :] = jnp.dot(x_ref[...], w_ref[...],
                            preferred_element_type=jnp.float32)
    y_ref[N:NP, :] = jnp.zeros((NP - N, D_HID), jnp.float32)


def _tcmm(x, W1):
    return pl.pallas_call(
        _tcmm_body,
        out_shape=jax.ShapeDtypeStruct((NP, D_HID), jnp.float32),
    )(x, W1)


def _tcscale_body(y_ref, d0_ref, d1_ref, dinv_ref, ht_ref):
    deg = d0_ref[:, 0:1] + d1_ref[:, 0:1] - 1.0
    dinv = lax.rsqrt(deg)
    dinv_ref[...] = dinv
    ht_ref[...] = dinv * y_ref[...]


def _tcscale(y, d):
    return pl.pallas_call(
        _tcscale_body,
        out_shape=[
            jax.ShapeDtypeStruct((NP, 1), jnp.float32),
            jax.ShapeDtypeStruct((NP, D_HID), jnp.float32),
        ],
    )(y, d[0], d[1])


def _comb_body(p_ref, ht_ref, dinv_ref, b_ref, w_ref, out_ref):
    dinv = dinv_ref[...]
    t = p_ref[0] + p_ref[1] - ht_ref[...]
    h = jnp.maximum(dinv * t + b_ref[...], 0.0)
    out_ref[...] = dinv * jnp.dot(
        h, w_ref[...], preferred_element_type=jnp.float32
    )


def _comb(p, ht, dinv, b, W, D_next):
    return pl.pallas_call(
        _comb_body,
        out_shape=jax.ShapeDtypeStruct((NP, D_next), jnp.float32),
    )(p, ht, dinv, b, W)


def _final_body(p_ref, ht_ref, dinv_ref, b_ref, out_ref):
    dinv = dinv_ref[...]
    t = p_ref[0] + p_ref[1] - ht_ref[...]
    h = jnp.maximum(dinv * t + b_ref[...], 0.0)
    out_ref[...] = jnp.sum(h[0:N], axis=0, keepdims=True) * (1.0 / N)


def _final(p, ht, dinv, b):
    return pl.pallas_call(
        _final_body,
        out_shape=jax.ShapeDtypeStruct((1, D_OUT), jnp.float32),
    )(p, ht, dinv, b)


def kernel(x, edge_index, last_rej_rate, W1, b1, W2, b2, W3, b3):
    pad_dst = N + jnp.arange(E_PAD - E, dtype=jnp.int32) % (NP - N)
    pad_src = N + (jnp.arange(E_PAD - E, dtype=jnp.int32) + 7) % (NP - N)
    srcp = jnp.concatenate([edge_index[0], pad_src])
    dstp = jnp.concatenate([edge_index[1], pad_dst])
    d = _get_deg()(dstp)
    y1 = _tcmm(x, W1)
    dinv, ht1 = _tcscale(y1, d)
    p1 = _get_agg(D_HID)(srcp, dstp, ht1)
    ht2 = _comb(p1, ht1, dinv, b1.reshape(1, -1), W2, D_HID)
    p2 = _get_agg(D_HID)(srcp, dstp, ht2)
    ht3 = _comb(p2, ht2, dinv, b2.reshape(1, -1), W3, D_OUT)
    p3 = _get_agg(D_OUT)(srcp, dstp, ht3)
    pooled = _final(p3, ht3, dinv, b3.reshape(1, -1))
    rej = jnp.reshape(last_rej_rate, (1, 1)).astype(jnp.float32)
    return jnp.concatenate([pooled, rej], axis=-1)
